# Initial kernel scaffold; baseline (speedup 1.0000x reference)
import jax
import jax.numpy as jnp
from jax.experimental import pallas as pl


def kernel(x, emb_table, pos_table):
    raise NotImplementedError("write your pallas kernel here")


# speedup vs baseline: 3.0987x; 3.0987x over previous
import functools

import jax
import jax.numpy as jnp
from jax import lax
from jax.experimental import pallas as pl
from jax.experimental.pallas import tpu as pltpu
from jax.experimental.pallas import tpu_sc as plsc

BATCH = 4096
SEQLEN = 200
NUM_HID = 64
NC = 2
NS = 16
NW = NC * NS
NB = BATCH // NW
IDX_SPLIT = 2
IDX_CHUNK = SEQLEN // IDX_SPLIT
LANES = 16
HID_VECS = NUM_HID // LANES


def _make_kernel():
    mesh = plsc.VectorSubcoreMesh(core_axis_name="c", subcore_axis_name="s")

    @functools.partial(
        pl.kernel,
        out_type=jax.ShapeDtypeStruct((BATCH, SEQLEN, NUM_HID), jnp.float32),
        mesh=mesh,
        scratch_types=[
            pltpu.VMEM((IDX_SPLIT, IDX_CHUNK), jnp.int32),
            pltpu.VMEM((SEQLEN, NUM_HID), jnp.float32),
            pltpu.VMEM((SEQLEN, NUM_HID), jnp.float32),
            pltpu.SemaphoreType.DMA,
            pltpu.SemaphoreType.DMA,
        ],
        compiler_params=pltpu.CompilerParams(use_tc_tiling_on_sc=False),
    )
    def k(x_hbm, emb_hbm, pos_hbm, out_hbm, idx_v, rows_v, pos_v, sem0, sem1):
        wid = lax.axis_index("s") * NC + lax.axis_index("c")
        pltpu.sync_copy(pos_hbm, pos_v)

        def step(i, carry):
            b = wid * NB + i
            pltpu.sync_copy(x_hbm.at[b], idx_v)
            cp0 = pltpu.async_copy(
                emb_hbm.at[idx_v.at[0]], rows_v.at[pl.ds(0, IDX_CHUNK)], sem0)
            cp1 = pltpu.async_copy(
                emb_hbm.at[idx_v.at[1]],
                rows_v.at[pl.ds(IDX_CHUNK, IDX_CHUNK)], sem1)
            cp0.wait()
            cp1.wait()

            def add_row(r, carry2):
                for cc in range(HID_VECS):
                    sl = pl.ds(cc * LANES, LANES)
                    plsc.addupdate(rows_v.at[r, sl], pos_v[r, sl])
                return carry2

            lax.fori_loop(0, SEQLEN, add_row, 0)
            pltpu.sync_copy(rows_v, out_hbm.at[b])
            return carry

        lax.fori_loop(0, NB, step, 0)

    return k


_kernel_call = _make_kernel()


def kernel(x, emb_table, pos_table):
    x3 = x.reshape(BATCH, IDX_SPLIT, IDX_CHUNK)
    return _kernel_call(x3, emb_table, pos_table)
